# baseline (speedup 1.0000x reference)
import functools

import jax
import jax.numpy as jnp
from jax import lax
from jax.experimental import pallas as pl
from jax.experimental.pallas import tpu as pltpu
from jax.experimental.pallas import tpu_sc as plsc

N = 10000
E = 320000
D = 128
L = 3

NC = 2
NS = 16
NW = NC * NS
CB = 64
NCH = E // CB
CHUNKS = 160
CQ = 40
NPH = CHUNKS // CQ
RB = 4
G = 2
N_PAD = 10240
RPT = N_PAD // NS
LAST_RPT = N - (NS - 1) * RPT


@functools.lru_cache(maxsize=None)
def _make_sc_segment_sum():
    mesh = plsc.VectorSubcoreMesh(core_axis_name="c", subcore_axis_name="s")

    @functools.partial(
        pl.kernel,
        out_type=jax.ShapeDtypeStruct((NC * N, D), jnp.float32),
        mesh=mesh,
        scratch_types=[
            pltpu.VMEM((CQ, CB), jnp.int32),
            pltpu.VMEM((CQ, CB), jnp.int32),
            pltpu.VMEM((RB * CB, D), jnp.float32),
            pltpu.VMEM_SHARED((N_PAD, D), jnp.float32),
            [pltpu.SemaphoreType.DMA] * RB,
            [pltpu.SemaphoreType.DMA] * RB,
        ],
    )
    def body(h_hbm, e_hbm, out_hbm, sidx, didx, ring, acc, gsems, ssems):
        c = lax.axis_index("c")
        s = lax.axis_index("s")
        w = c * NS + s

        def zrow(i, carry):
            for j in range(D // 16):
                ring[i, pl.ds(j * 16, 16)] = jnp.zeros((16,), jnp.float32)
            return carry

        lax.fori_loop(0, RB * CB, zrow, 0)
        for k in range(RPT // (RB * CB)):
            pltpu.sync_copy(ring, acc.at[pl.ds(s * RPT + k * RB * CB, RB * CB)])
        rem = RPT % (RB * CB)
        if rem:
            pltpu.sync_copy(ring.at[pl.ds(0, rem)],
                            acc.at[pl.ds(s * RPT + RPT - rem, rem)])
        plsc.subcore_barrier()

        nph = jnp.where(w == NW - 1, 1, NPH)

        def buf(j):
            return ring.at[pl.ds(j * CB, CB)]

        def fire_gather(k, j):
            pltpu.async_copy(h_hbm.at[sidx.at[k]], buf(j), gsems[j])

        def wait_gather(j):
            pltpu.make_async_copy(h_hbm.at[pl.ds(0, CB)], buf(j),
                                  gsems[j]).wait()

        def fire_scatter(k, j):
            pltpu.async_copy(buf(j), acc.at[didx.at[k]], ssems[j], add=True)

        def wait_scatter(j):
            pltpu.make_async_copy(buf(j), acc.at[didx.at[0]], ssems[j]).wait()

        def phase(p, carry):
            @pl.when(p > 0)
            def _():
                for j in range(G, RB):
                    wait_scatter(j)

            base = w * CHUNKS + p * CQ
            pltpu.sync_copy(e_hbm.at[0, pl.ds(base, CQ)], sidx)
            pltpu.sync_copy(e_hbm.at[1, pl.ds(base, CQ)], didx)
            for j in range(G):
                fire_gather(j, j)

            def rounds(q, carry2):
                for j in range(RB):
                    k = RB * q + j
                    jg = (j + G) % RB
                    if j < RB - G:
                        @pl.when(q >= 1)
                        def _():
                            wait_scatter(jg)

                        fire_gather(k + G, jg)
                    else:
                        wait_scatter(jg)

                        @pl.when(q < CQ // RB - 1)
                        def _():
                            fire_gather(k + G, jg)

                    wait_gather(j)
                    fire_scatter(k, j)
                return carry2

            lax.fori_loop(0, CQ // RB, rounds, 0)
            return carry

        lax.fori_loop(0, nph, phase, 0)
        for j in range(G, RB):
            wait_scatter(j)
        plsc.subcore_barrier()

        @pl.when(s < NS - 1)
        def _():
            pltpu.sync_copy(acc.at[pl.ds(s * RPT, RPT)],
                            out_hbm.at[pl.ds(c * N + s * RPT, RPT)])

        @pl.when(s == NS - 1)
        def _():
            pltpu.sync_copy(acc.at[pl.ds((NS - 1) * RPT, LAST_RPT)],
                            out_hbm.at[pl.ds(c * N + (NS - 1) * RPT, LAST_RPT)])

    return body


def _sc_segment_sum(h, e3):
    return _make_sc_segment_sum()(h, e3)


@functools.lru_cache(maxsize=None)
def _make_tc_layer():
    def body(h_ref, a0_ref, a1_ref, ws_ref, wn_ref, b_ref, out_ref):
        hblk = h_ref[...]
        out = jnp.dot(hblk, ws_ref[...], preferred_element_type=jnp.float32)
        asum = a0_ref[...] + a1_ref[...]
        out += jnp.dot(asum, wn_ref[...], preferred_element_type=jnp.float32)
        out += b_ref[...]
        out_ref[...] = jnp.maximum(out, 0.0) + hblk

    blk = 1000
    nblk = N // blk
    return pl.pallas_call(
        body,
        grid=(nblk,),
        in_specs=[
            pl.BlockSpec((blk, D), lambda i: (i, 0)),
            pl.BlockSpec((blk, D), lambda i: (i, 0)),
            pl.BlockSpec((blk, D), lambda i: (i + nblk, 0)),
            pl.BlockSpec((D, D), lambda i: (0, 0)),
            pl.BlockSpec((D, D), lambda i: (0, 0)),
            pl.BlockSpec((1, D), lambda i: (0, 0)),
        ],
        out_specs=pl.BlockSpec((blk, D), lambda i: (i, 0)),
        out_shape=jax.ShapeDtypeStruct((N, D), jnp.float32),
    )


def _tc_layer(h, a0, a1, w_self, w_nbr, bias):
    return _make_tc_layer()(h, a0, a1, w_self, w_nbr, bias)


def kernel(x, edge_index, W_self, W_nbr, b):
    e3 = edge_index.reshape(2, NCH, CB)

    h = x
    for i in range(L):
        agg = _sc_segment_sum(h, e3)
        h = _tc_layer(h, agg, agg, W_self[i], W_nbr[i], b[i].reshape(1, D))
    return h


# speedup vs baseline: 1.0125x; 1.0125x over previous
import functools

import jax
import jax.numpy as jnp
from jax import lax
from jax.experimental import pallas as pl
from jax.experimental.pallas import tpu as pltpu
from jax.experimental.pallas import tpu_sc as plsc

N = 10000
E = 320000
D = 128
L = 3

NC = 2
NS = 16
NW = NC * NS
CB = 64
NCH = E // CB
CHUNKS = 160
CQ = 40
NPH = CHUNKS // CQ
RB = 4
G = 2
N_PAD = 10240
RPT = N_PAD // NS
LAST_RPT = N - (NS - 1) * RPT


@functools.lru_cache(maxsize=None)
def _make_sc_segment_sum():
    mesh = plsc.VectorSubcoreMesh(core_axis_name="c", subcore_axis_name="s")

    @functools.partial(
        pl.kernel,
        out_type=jax.ShapeDtypeStruct((NC * N, D), jnp.float32),
        mesh=mesh,
        scratch_types=[
            pltpu.VMEM((CQ, CB), jnp.int32),
            pltpu.VMEM((CQ, CB), jnp.int32),
            pltpu.VMEM((RB * CB, D), jnp.float32),
            pltpu.VMEM_SHARED((N_PAD, D), jnp.float32),
            [pltpu.SemaphoreType.DMA] * RB,
            [pltpu.SemaphoreType.DMA] * RB,
        ],
    )
    def body(h_hbm, e_hbm, out_hbm, sidx, didx, ring, acc, gsems, ssems):
        c = lax.axis_index("c")
        s = lax.axis_index("s")
        w = c * NS + s

        nph = jnp.where(w == NW - 1, 1, NPH)

        def buf(j):
            return ring.at[pl.ds(j * CB, CB)]

        def fire_gather(k, j):
            pltpu.async_copy(h_hbm.at[sidx.at[k]], buf(j), gsems[j])

        def wait_gather(j):
            pltpu.make_async_copy(h_hbm.at[pl.ds(0, CB)], buf(j),
                                  gsems[j]).wait()

        def fire_scatter(k, j):
            pltpu.async_copy(buf(j), acc.at[didx.at[k]], ssems[j], add=True)

        def wait_scatter(j):
            pltpu.make_async_copy(buf(j), acc.at[didx.at[0]], ssems[j]).wait()

        base0 = w * CHUNKS
        pltpu.sync_copy(e_hbm.at[0, pl.ds(base0, CQ)], sidx)
        pltpu.sync_copy(e_hbm.at[1, pl.ds(base0, CQ)], didx)
        for j in range(G):
            fire_gather(j, j)

        ZR = (RB - G) * CB

        def zrow(i, carry):
            for j in range(D // 16):
                ring[G * CB + i, pl.ds(j * 16, 16)] = jnp.zeros((16,),
                                                               jnp.float32)
            return carry

        lax.fori_loop(0, ZR, zrow, 0)
        for k in range(RPT // ZR):
            pltpu.sync_copy(ring.at[pl.ds(G * CB, ZR)],
                            acc.at[pl.ds(s * RPT + k * ZR, ZR)])
        rem = RPT % ZR
        if rem:
            pltpu.sync_copy(ring.at[pl.ds(G * CB, rem)],
                            acc.at[pl.ds(s * RPT + RPT - rem, rem)])
        plsc.subcore_barrier()

        def phase(p, carry):
            @pl.when(p > 0)
            def _():
                for j in range(G, RB):
                    wait_scatter(j)
                base = w * CHUNKS + p * CQ
                pltpu.sync_copy(e_hbm.at[0, pl.ds(base, CQ)], sidx)
                pltpu.sync_copy(e_hbm.at[1, pl.ds(base, CQ)], didx)
                for j in range(G):
                    fire_gather(j, j)

            def rounds(q, carry2):
                for j in range(RB):
                    k = RB * q + j
                    jg = (j + G) % RB
                    if j < RB - G:
                        @pl.when(q >= 1)
                        def _():
                            wait_scatter(jg)

                        fire_gather(k + G, jg)
                    else:
                        wait_scatter(jg)

                        @pl.when(q < CQ // RB - 1)
                        def _():
                            fire_gather(k + G, jg)

                    wait_gather(j)
                    fire_scatter(k, j)
                return carry2

            lax.fori_loop(0, CQ // RB, rounds, 0)
            return carry

        lax.fori_loop(0, nph, phase, 0)
        for j in range(G, RB):
            wait_scatter(j)
        plsc.subcore_barrier()

        @pl.when(s < NS - 1)
        def _():
            pltpu.sync_copy(acc.at[pl.ds(s * RPT, RPT)],
                            out_hbm.at[pl.ds(c * N + s * RPT, RPT)])

        @pl.when(s == NS - 1)
        def _():
            pltpu.sync_copy(acc.at[pl.ds((NS - 1) * RPT, LAST_RPT)],
                            out_hbm.at[pl.ds(c * N + (NS - 1) * RPT, LAST_RPT)])

    return body


def _sc_segment_sum(h, e3):
    return _make_sc_segment_sum()(h, e3)


@functools.lru_cache(maxsize=None)
def _make_tc_layer():
    def body(h_ref, a0_ref, a1_ref, ws_ref, wn_ref, b_ref, out_ref):
        hblk = h_ref[...]
        out = jnp.dot(hblk, ws_ref[...], preferred_element_type=jnp.float32)
        asum = a0_ref[...] + a1_ref[...]
        out += jnp.dot(asum, wn_ref[...], preferred_element_type=jnp.float32)
        out += b_ref[...]
        out_ref[...] = jnp.maximum(out, 0.0) + hblk

    blk = 1000
    nblk = N // blk
    return pl.pallas_call(
        body,
        grid=(nblk,),
        in_specs=[
            pl.BlockSpec((blk, D), lambda i: (i, 0)),
            pl.BlockSpec((blk, D), lambda i: (i, 0)),
            pl.BlockSpec((blk, D), lambda i: (i + nblk, 0)),
            pl.BlockSpec((D, D), lambda i: (0, 0)),
            pl.BlockSpec((D, D), lambda i: (0, 0)),
            pl.BlockSpec((1, D), lambda i: (0, 0)),
        ],
        out_specs=pl.BlockSpec((blk, D), lambda i: (i, 0)),
        out_shape=jax.ShapeDtypeStruct((N, D), jnp.float32),
    )


def _tc_layer(h, a0, a1, w_self, w_nbr, bias):
    return _make_tc_layer()(h, a0, a1, w_self, w_nbr, bias)


def kernel(x, edge_index, W_self, W_nbr, b):
    e3 = edge_index.reshape(2, NCH, CB)

    h = x
    for i in range(L):
        agg = _sc_segment_sum(h, e3)
        h = _tc_layer(h, agg, agg, W_self[i], W_nbr[i], b[i].reshape(1, D))
    return h
